# baseline (speedup 1.0000x reference)
import jax
import jax.numpy as jnp
from jax.experimental import pallas as pl

EPS = 1e-05
LOG2E = 1.4426950408889634
BLOCK_B = 16384


def _body(x_ref, c_ref, w_ref, o_ref):
    x = x_ref[...]
    c = c_ref[...]
    d = LOG2E / (jnp.abs(x - c) + EPS)
    m = jnp.max(d, axis=1, keepdims=True)
    e = jnp.exp2(d - m)
    vs = jnp.dot(e, w_ref[...], preferred_element_type=jnp.float32)
    embed_dim = vs.shape[1] - 1
    o_ref[...] = vs[:, :embed_dim] / vs[:, embed_dim:]


def kernel(x, emb_weight, centroid):
    batch = x.shape[0]
    num_level, embed_dim = emb_weight.shape
    c_row = centroid.reshape(1, num_level)
    w_aug = jnp.concatenate(
        [emb_weight, jnp.ones((num_level, 1), jnp.float32)], axis=1)
    grid = batch // BLOCK_B
    return pl.pallas_call(
        _body,
        grid=(grid,),
        in_specs=[
            pl.BlockSpec((BLOCK_B, 1), lambda i: (i, 0)),
            pl.BlockSpec((1, num_level), lambda i: (0, 0)),
            pl.BlockSpec((num_level, embed_dim + 1), lambda i: (0, 0)),
        ],
        out_specs=pl.BlockSpec((BLOCK_B, embed_dim), lambda i: (i, 0)),
        out_shape=jax.ShapeDtypeStruct((batch, embed_dim), jnp.float32),
    )(x, c_row, w_aug)


# speedup vs baseline: 1.6697x; 1.6697x over previous
import jax
import jax.numpy as jnp
from jax.experimental import pallas as pl

EPS = 1e-05
LOG2E = 1.4426950408889634
CAP = 80.0
BLOCK_B = 4096


def _body(x_ref, c_ref, wt_ref, o_ref):
    x = x_ref[...]
    c = c_ref[...]
    d = LOG2E / (jnp.abs(x - c) + EPS)
    u = jnp.exp2(jnp.minimum(d, CAP))
    vs = jnp.dot(wt_ref[...], u, preferred_element_type=jnp.float32)
    embed_dim = vs.shape[0] - 1
    ot = vs[:embed_dim, :] * (1.0 / vs[embed_dim:, :])
    o_ref[...] = ot.T


def kernel(x, emb_weight, centroid):
    batch = x.shape[0]
    num_level, embed_dim = emb_weight.shape
    x_row = x.reshape(1, batch)
    w_aug_t = jnp.concatenate(
        [emb_weight.T, jnp.ones((1, num_level), jnp.float32)], axis=0)
    grid = batch // BLOCK_B
    return pl.pallas_call(
        _body,
        grid=(grid,),
        in_specs=[
            pl.BlockSpec((1, BLOCK_B), lambda i: (0, i)),
            pl.BlockSpec((num_level, 1), lambda i: (0, 0)),
            pl.BlockSpec((embed_dim + 1, num_level), lambda i: (0, 0)),
        ],
        out_specs=pl.BlockSpec((BLOCK_B, embed_dim), lambda i: (i, 0)),
        out_shape=jax.ShapeDtypeStruct((batch, embed_dim), jnp.float32),
    )(x_row, centroid, w_aug_t)


# speedup vs baseline: 4.0152x; 2.4048x over previous
import jax
import jax.numpy as jnp
from jax.experimental import pallas as pl

EPS = 1e-05
LOG2E = 1.4426950408889634
CAP = 80.0
BLOCK_B = 4096


def _body(x_ref, c_ref, wt_ref, o_ref):
    x = x_ref[...]
    c = c_ref[...]
    d = LOG2E / (jnp.abs(x - c) + EPS)
    u = jnp.exp2(jnp.minimum(d, CAP))
    vs = jnp.dot(wt_ref[...], u, preferred_element_type=jnp.float32)
    embed_dim = vs.shape[0] - 1
    o_ref[...] = vs[:embed_dim, :] * (1.0 / vs[embed_dim:, :])


def kernel(x, emb_weight, centroid):
    batch = x.shape[0]
    num_level, embed_dim = emb_weight.shape
    x_row = x.reshape(1, batch)
    w_aug_t = jnp.concatenate(
        [emb_weight.T, jnp.ones((1, num_level), jnp.float32)], axis=0)
    grid = batch // BLOCK_B
    out_t = pl.pallas_call(
        _body,
        grid=(grid,),
        in_specs=[
            pl.BlockSpec((1, BLOCK_B), lambda i: (0, i)),
            pl.BlockSpec((num_level, 1), lambda i: (0, 0)),
            pl.BlockSpec((embed_dim + 1, num_level), lambda i: (0, 0)),
        ],
        out_specs=pl.BlockSpec((embed_dim, BLOCK_B), lambda i: (0, i)),
        out_shape=jax.ShapeDtypeStruct((embed_dim, batch), jnp.float32),
    )(x_row, centroid, w_aug_t)
    return out_t.T
